# Initial kernel scaffold; baseline (speedup 1.0000x reference)
import jax
import jax.numpy as jnp
from jax.experimental import pallas as pl


def kernel(x, W, b):
    raise NotImplementedError("write your pallas kernel here")


# speedup vs baseline: 1.3835x; 1.3835x over previous
import functools

import jax
import jax.numpy as jnp
from jax.experimental import pallas as pl
from jax.experimental.pallas import tpu as pltpu

DIM_ = 4096
E_ = 64
K_ = 8
BLK = 512


def _gate_kernel(x_ref, w_ref, b_ref, idx_ref, val_ref, aux_ref,
                 imp_acc, load_acc, *, nsteps, total_rows):
    step = pl.program_id(0)

    logits = jax.lax.dot_general(
        x_ref[...], w_ref[...],
        dimension_numbers=(((1,), (1,)), ((), ())),
        preferred_element_type=jnp.float32,
    ) + b_ref[...]

    m = jnp.max(logits, axis=1, keepdims=True)
    e = jnp.exp(logits - m)
    denom = jnp.sum(e, axis=1, keepdims=True)
    probs = e / denom

    lane = jax.lax.broadcasted_iota(jnp.int32, (BLK, E_), 1)

    cur = probs
    for k in range(K_):
        mk = jnp.max(cur, axis=1, keepdims=True)
        ik = jnp.min(jnp.where(cur == mk, lane, E_), axis=1, keepdims=True)
        val_ref[:, k] = mk[:, 0]
        idx_ref[:, k] = ik[:, 0]
        if k == 0:
            hard1 = ik
        if k != K_ - 1:
            cur = jnp.where(lane == ik, -1.0, cur)

    @pl.when(step == 0)
    def _init():
        imp_acc[...] = jnp.zeros_like(imp_acc)
        load_acc[...] = jnp.zeros_like(load_acc)

    imp_acc[...] += jnp.sum(probs, axis=0, keepdims=True)
    onehot = (lane == hard1).astype(jnp.float32)
    load_acc[...] += jnp.sum(onehot, axis=0, keepdims=True)

    @pl.when(step == nsteps - 1)
    def _finalize():
        inv_s = 1.0 / total_rows
        importance = imp_acc[...] * inv_s
        load = jnp.maximum(load_acc[...] * inv_s, 1e-9)
        aux_ref[...] = E_ * jnp.sum(importance * load, keepdims=True)


def kernel(x, W, b):
    S = x.shape[0]
    nsteps = S // BLK
    b2 = b.reshape(1, E_)

    idx, vals, aux = pl.pallas_call(
        functools.partial(_gate_kernel, nsteps=nsteps, total_rows=float(S)),
        grid=(nsteps,),
        in_specs=[
            pl.BlockSpec((BLK, DIM_), lambda i: (i, 0)),
            pl.BlockSpec((E_, DIM_), lambda i: (0, 0)),
            pl.BlockSpec((1, E_), lambda i: (0, 0)),
        ],
        out_specs=[
            pl.BlockSpec((BLK, K_), lambda i: (i, 0)),
            pl.BlockSpec((BLK, K_), lambda i: (i, 0)),
            pl.BlockSpec((1, 1), lambda i: (0, 0)),
        ],
        out_shape=[
            jax.ShapeDtypeStruct((S, K_), jnp.int32),
            jax.ShapeDtypeStruct((S, K_), jnp.float32),
            jax.ShapeDtypeStruct((1, 1), jnp.float32),
        ],
        scratch_shapes=[
            pltpu.VMEM((1, E_), jnp.float32),
            pltpu.VMEM((1, E_), jnp.float32),
        ],
    )(x, W, b2)

    return idx, vals, aux.reshape(())


# speedup vs baseline: 1.6100x; 1.1638x over previous
import functools

import jax
import jax.numpy as jnp
from jax.experimental import pallas as pl
from jax.experimental.pallas import tpu as pltpu

DIM_ = 4096
E_ = 64
K_ = 8
BLK = 1024


def _gate_kernel(x_ref, w_ref, b_ref, idx_ref, val_ref, aux_ref,
                 imp_acc, load_acc, *, nsteps, total_rows):
    step = pl.program_id(0)

    logits = jax.lax.dot_general(
        x_ref[...], w_ref[...],
        dimension_numbers=(((1,), (1,)), ((), ())),
        preferred_element_type=jnp.float32,
    ) + b_ref[...]

    m = jnp.max(logits, axis=1, keepdims=True)
    e = jnp.exp(logits - m)
    denom = jnp.sum(e, axis=1, keepdims=True)
    probs = e / denom

    lane = jax.lax.broadcasted_iota(jnp.int32, (BLK, E_), 1)

    cur = probs
    for k in range(K_):
        mk = jnp.max(cur, axis=1, keepdims=True)
        ik = jnp.min(jnp.where(cur == mk, lane, E_), axis=1, keepdims=True)
        val_ref[:, k] = mk[:, 0]
        idx_ref[:, k] = ik[:, 0]
        if k == 0:
            hard1 = ik
        if k != K_ - 1:
            cur = jnp.where(lane == ik, -1.0, cur)

    @pl.when(step == 0)
    def _init():
        imp_acc[...] = jnp.zeros_like(imp_acc)
        load_acc[...] = jnp.zeros_like(load_acc)

    imp_acc[...] += jnp.sum(probs, axis=0, keepdims=True)
    onehot = (lane == hard1).astype(jnp.float32)
    load_acc[...] += jnp.sum(onehot, axis=0, keepdims=True)

    @pl.when(step == nsteps - 1)
    def _finalize():
        inv_s = 1.0 / total_rows
        importance = imp_acc[...] * inv_s
        load = jnp.maximum(load_acc[...] * inv_s, 1e-9)
        aux_ref[...] = E_ * jnp.sum(importance * load, keepdims=True)


def kernel(x, W, b):
    S = x.shape[0]
    nsteps = S // BLK
    b2 = b.reshape(1, E_)

    idx, vals, aux = pl.pallas_call(
        functools.partial(_gate_kernel, nsteps=nsteps, total_rows=float(S)),
        grid=(nsteps,),
        in_specs=[
            pl.BlockSpec((BLK, DIM_), lambda i: (i, 0)),
            pl.BlockSpec((E_, DIM_), lambda i: (0, 0)),
            pl.BlockSpec((1, E_), lambda i: (0, 0)),
        ],
        out_specs=[
            pl.BlockSpec((BLK, K_), lambda i: (i, 0)),
            pl.BlockSpec((BLK, K_), lambda i: (i, 0)),
            pl.BlockSpec((1, 1), lambda i: (0, 0)),
        ],
        out_shape=[
            jax.ShapeDtypeStruct((S, K_), jnp.int32),
            jax.ShapeDtypeStruct((S, K_), jnp.float32),
            jax.ShapeDtypeStruct((1, 1), jnp.float32),
        ],
        scratch_shapes=[
            pltpu.VMEM((1, E_), jnp.float32),
            pltpu.VMEM((1, E_), jnp.float32),
        ],
    )(x, W, b2)

    return idx, vals, aux.reshape(())


# speedup vs baseline: 1.8855x; 1.1711x over previous
import functools

import jax
import jax.numpy as jnp
from jax.experimental import pallas as pl
from jax.experimental.pallas import tpu as pltpu

DIM_ = 4096
E_ = 64
K_ = 8
BLK = 1024


def _gate_kernel(x_ref, w_ref, b_ref, idx_ref, val_ref, aux_ref,
                 imp_acc, load_acc, *, nsteps, total_rows):
    step = pl.program_id(0)

    logits = jax.lax.dot_general(
        w_ref[...], x_ref[...],
        dimension_numbers=(((1,), (1,)), ((), ())),
        preferred_element_type=jnp.float32,
    ) + b_ref[...]

    m = jnp.max(logits, axis=0, keepdims=True)
    e = jnp.exp(logits - m)
    denom = jnp.sum(e, axis=0, keepdims=True)
    probs = e / denom

    expert = jax.lax.broadcasted_iota(jnp.int32, (E_, BLK), 0)

    cur = probs
    vals = []
    idxs = []
    for k in range(K_):
        mk = jnp.max(cur, axis=0, keepdims=True)
        ik = jnp.min(jnp.where(cur == mk, expert, E_), axis=0, keepdims=True)
        vals.append(mk)
        idxs.append(ik)
        if k == 0:
            hard1 = ik
        if k != K_ - 1:
            cur = jnp.where(expert == ik, -1.0, cur)

    valsT = jnp.concatenate(vals, axis=0)
    idxsT = jnp.concatenate(idxs, axis=0)
    val_ref[...] = valsT.T
    idx_ref[...] = idxsT.T

    @pl.when(step == 0)
    def _init():
        imp_acc[...] = jnp.zeros_like(imp_acc)
        load_acc[...] = jnp.zeros_like(load_acc)

    imp_acc[...] += probs
    load_acc[...] += (expert == hard1).astype(jnp.float32)

    @pl.when(step == nsteps - 1)
    def _finalize():
        inv_s = 1.0 / total_rows
        importance = jnp.sum(imp_acc[...], axis=1) * inv_s
        load = jnp.maximum(jnp.sum(load_acc[...], axis=1) * inv_s, 1e-9)
        aux_ref[...] = E_ * jnp.sum(importance * load, keepdims=True).reshape(1, 1)


def kernel(x, W, b):
    S = x.shape[0]
    nsteps = S // BLK
    b2 = b.reshape(E_, 1)

    idx, vals, aux = pl.pallas_call(
        functools.partial(_gate_kernel, nsteps=nsteps, total_rows=float(S)),
        grid=(nsteps,),
        in_specs=[
            pl.BlockSpec((BLK, DIM_), lambda i: (i, 0)),
            pl.BlockSpec((E_, DIM_), lambda i: (0, 0)),
            pl.BlockSpec((E_, 1), lambda i: (0, 0)),
        ],
        out_specs=[
            pl.BlockSpec((BLK, K_), lambda i: (i, 0)),
            pl.BlockSpec((BLK, K_), lambda i: (i, 0)),
            pl.BlockSpec((1, 1), lambda i: (0, 0)),
        ],
        out_shape=[
            jax.ShapeDtypeStruct((S, K_), jnp.int32),
            jax.ShapeDtypeStruct((S, K_), jnp.float32),
            jax.ShapeDtypeStruct((1, 1), jnp.float32),
        ],
        scratch_shapes=[
            pltpu.VMEM((E_, BLK), jnp.float32),
            pltpu.VMEM((E_, BLK), jnp.float32),
        ],
    )(x, W, b2)

    return idx, vals, aux.reshape(())
